# baseline (speedup 1.0000x reference)
import functools

import jax
import jax.numpy as jnp
from jax import lax
from jax.experimental import pallas as pl
from jax.experimental.pallas import tpu as pltpu
from jax.experimental.pallas import tpu_sc as plsc

_R, _C = 32, 128


def _scores_body(f_ref, s_ref):
    x = f_ref[0]
    c = x * x
    acc = c[:, 0:128]
    for q in range(1, 8):
        acc = acc + c[:, 128 * q:128 * (q + 1)]
    s = acc
    for i in range(1, 16):
        s = s + pltpu.roll(acc, 128 - 8 * i, 1)
    u = s[:, 0:4] + s[:, 4:8]
    v = u[:, 0:2] + u[:, 2:4]
    w = v[:, 0:1] + v[:, 1:2]
    s_ref[0] = jnp.sqrt(w)


def _scores(feat, tblk):
    b, n, c = feat.shape
    return pl.pallas_call(
        _scores_body,
        grid=(b, n // tblk),
        in_specs=[pl.BlockSpec((1, tblk, c), lambda i, t: (i, t, 0))],
        out_specs=pl.BlockSpec((1, tblk, 1), lambda i, t: (i, t, 0)),
        out_shape=jax.ShapeDtypeStruct((b, n, 1), jnp.float32),
    )(feat)


def _xor_lane(x, j):
    lo = pltpu.roll(x, _C - j, x.ndim - 1)
    hi = pltpu.roll(x, j, x.ndim - 1)
    bit = (lax.broadcasted_iota(jnp.int32, x.shape, x.ndim - 1) & j) == 0
    return jnp.where(bit, lo, hi)


def _xor_row(x, m):
    b = x.shape[0]
    x5 = x.reshape(b, _R // (2 * m), 2, m, _C)
    sw = jnp.concatenate([x5[:, :, 1:2], x5[:, :, 0:1]], axis=2)
    return sw.reshape(b, _R, _C)


def _sort_stage(key, idx, K, j):
    if j < _C:
        kp = _xor_lane(key, j)
        xp = _xor_lane(idx, j)
        lower = (lax.broadcasted_iota(jnp.int32, key.shape, 2) & j) == 0
    else:
        m = j // _C
        kp = _xor_row(key, m)
        xp = _xor_row(idx, m)
        lower = (lax.broadcasted_iota(jnp.int32, key.shape, 1) & m) == 0
    if K < _C:
        dird = (lax.broadcasted_iota(jnp.int32, key.shape, 2) & K) == 0
    else:
        dird = (lax.broadcasted_iota(jnp.int32, key.shape, 1) & (K // _C)) == 0
    cmp = (key > kp) | ((key == kp) & (idx < xp))
    take_own = cmp == (lower == dird)
    return jnp.where(take_own, key, kp), jnp.where(take_own, idx, xp)


def _sort_body(s_ref, si_ref):
    key = s_ref[...]
    idx = (lax.broadcasted_iota(jnp.int32, key.shape, 1) * _C +
           lax.broadcasted_iota(jnp.int32, key.shape, 2))
    for t in range(1, 13):
        K = 2 ** t
        j = K // 2
        while j >= 1:
            key, idx = _sort_stage(key, idx, K, j)
            j //= 2
    si_ref[...] = idx


def _sort(scores3):
    b = scores3.shape[0]
    return pl.pallas_call(
        _sort_body,
        in_specs=[pl.BlockSpec((b, _R, _C), lambda: (0, 0, 0))],
        out_specs=pl.BlockSpec((b, _R, _C), lambda: (0, 0, 0)),
        out_shape=jax.ShapeDtypeStruct((b, _R, _C), jnp.int32),
    )(scores3)


def _make_sc_invert(b, n):
    mesh = plsc.VectorSubcoreMesh(core_axis_name="c", subcore_axis_name="s")

    @functools.partial(
        pl.kernel,
        mesh=mesh,
        out_type=jax.ShapeDtypeStruct((b, n), jnp.int32),
        scratch_types=[
            pltpu.VMEM((n,), jnp.int32),
            pltpu.VMEM((n,), jnp.int32),
        ],
        compiler_params=pltpu.CompilerParams(needs_layout_passes=False),
    )
    def sc_invert(sidx_hbm, out_hbm, sidx_v, row_v):
        wid = lax.axis_index("s") * 2 + lax.axis_index("c")

        @pl.when(wid < b)
        def _():
            pltpu.sync_copy(sidx_hbm.at[wid], sidx_v)

            def body(v, carry):
                tok = sidx_v[pl.ds(v * 16, 16)]
                pos = lax.iota(jnp.int32, 16) + v * 16
                plsc.store_scatter(row_v, [tok], pos)
                return carry

            lax.fori_loop(0, n // 16, body, 0)
            pltpu.sync_copy(row_v, out_hbm.at[wid])

    return sc_invert


def _add_body(f_ref, r_ref, e_ref, o_ref, *, k):
    x = f_ref[0]
    r = r_ref[0]
    e = e_ref[0]
    o_ref[0] = x + jnp.where(r < k, 1.0, 0.0) * e


def _masked_add(feat, ranks_col, emb, k, tblk):
    b, n, c = feat.shape
    body = functools.partial(_add_body, k=k)
    return pl.pallas_call(
        body,
        grid=(b, n // tblk),
        in_specs=[
            pl.BlockSpec((1, tblk, c), lambda i, t: (i, t, 0)),
            pl.BlockSpec((1, tblk, 1), lambda i, t: (i, t, 0)),
            pl.BlockSpec((1, 1, c), lambda i, t: (0, 0, 0)),
        ],
        out_specs=pl.BlockSpec((1, tblk, c), lambda i, t: (i, t, 0)),
        out_shape=jax.ShapeDtypeStruct((b, n, c), jnp.float32),
    )(feat, ranks_col, emb)


def kernel(feat, emb):
    b, n, c = feat.shape
    k = int(0.4 * n)

    scores_col = _scores(feat, tblk=512)
    sorted_idx = _sort(scores_col.reshape(b, _R, _C))
    sorted_idx = sorted_idx.reshape(b, n)
    indices = sorted_idx[:, :k]
    ranks = _make_sc_invert(b, n)(sorted_idx)
    feat_out = _masked_add(feat, ranks.reshape(b, n, 1), emb, k, tblk=512)
    return (feat_out, indices)


# speedup vs baseline: 1.3662x; 1.3662x over previous
import functools

import jax
import jax.numpy as jnp
from jax import lax
from jax.experimental import pallas as pl
from jax.experimental.pallas import tpu as pltpu
from jax.experimental.pallas import tpu_sc as plsc

_R, _C = 32, 128


def _scores_sort_body(f_ref, si_ref, sc, *, tblk, nb, tb):
    bb = pl.program_id(0)
    t = pl.program_id(1)

    @pl.when(t < tb)
    def _sc():
        rows = []
        for i in range(tblk // 128):
            x = f_ref[0, pl.ds(128 * i, 128), :]
            c = x * x
            acc = c[:, 0:128]
            for q in range(1, 8):
                acc = acc + c[:, 128 * q:128 * (q + 1)]
            at = jnp.swapaxes(acc, 0, 1)
            s = at[0:8]
            for g in range(1, 16):
                s = s + at[8 * g:8 * (g + 1)]
            u = s[0:4] + s[4:8]
            v = u[0:2] + u[2:4]
            w = v[0:1] + v[1:2]
            rows.append(jnp.sqrt(w))
        sc[bb, pl.ds((tblk // 128) * t, tblk // 128)] = jnp.concatenate(
            rows, axis=0)

    @pl.when((t == tb) & (bb == nb - 1))
    def _srt():
        key = sc[...]
        idx = (lax.broadcasted_iota(jnp.int32, key.shape, 1) * _C +
               lax.broadcasted_iota(jnp.int32, key.shape, 2))
        for tt in range(1, 13):
            K = 2 ** tt
            j = K // 2
            while j >= 1:
                key, idx = _sort_stage(key, idx, K, j)
                j //= 2
        si_ref[...] = idx


def _scores_sort(feat, tblk):
    b, n, c = feat.shape
    tb = n // tblk
    body = functools.partial(_scores_sort_body, tblk=tblk, nb=b, tb=tb)
    return pl.pallas_call(
        body,
        grid=(b, tb + 1),
        in_specs=[pl.BlockSpec(
            (1, tblk, c),
            lambda i, t, tb=tb: (i, jnp.where(t < tb, t, tb - 1), 0))],
        out_specs=pl.BlockSpec((b, n // 128, 128), lambda i, t: (0, 0, 0)),
        out_shape=jax.ShapeDtypeStruct((b, n // 128, 128), jnp.int32),
        scratch_shapes=[pltpu.VMEM((b, n // 128, 128), jnp.float32)],
    )(feat)


def _scores_body(f_ref, s_ref, *, tblk):
    rows = []
    for i in range(tblk // 128):
        x = f_ref[0, pl.ds(128 * i, 128), :]
        c = x * x
        acc = c[:, 0:128]
        for q in range(1, 8):
            acc = acc + c[:, 128 * q:128 * (q + 1)]
        at = jnp.swapaxes(acc, 0, 1)
        s = at[0:8]
        for g in range(1, 16):
            s = s + at[8 * g:8 * (g + 1)]
        u = s[0:4] + s[4:8]
        v = u[0:2] + u[2:4]
        w = v[0:1] + v[1:2]
        rows.append(jnp.sqrt(w))
    s_ref[0] = jnp.concatenate(rows, axis=0)


def _scores(feat, tblk):
    b, n, c = feat.shape
    body = functools.partial(_scores_body, tblk=tblk)
    return pl.pallas_call(
        body,
        grid=(b, n // tblk),
        in_specs=[pl.BlockSpec((1, tblk, c), lambda i, t: (i, t, 0))],
        out_specs=pl.BlockSpec((1, tblk // 128, 128), lambda i, t: (i, t, 0)),
        out_shape=jax.ShapeDtypeStruct((b, n // 128, 128), jnp.float32),
    )(feat)


def _xor_lane(x, j):
    lo = pltpu.roll(x, _C - j, x.ndim - 1)
    hi = pltpu.roll(x, j, x.ndim - 1)
    bit = (lax.broadcasted_iota(jnp.int32, x.shape, x.ndim - 1) & j) == 0
    return jnp.where(bit, lo, hi)


def _xor_row(x, m):
    b = x.shape[0]
    x5 = x.reshape(b, _R // (2 * m), 2, m, _C)
    sw = jnp.concatenate([x5[:, :, 1:2], x5[:, :, 0:1]], axis=2)
    return sw.reshape(b, _R, _C)


def _sort_stage(key, idx, K, j):
    if j < _C:
        kp = _xor_lane(key, j)
        xp = _xor_lane(idx, j)
        lower = (lax.broadcasted_iota(jnp.int32, key.shape, 2) & j) == 0
    else:
        m = j // _C
        kp = _xor_row(key, m)
        xp = _xor_row(idx, m)
        lower = (lax.broadcasted_iota(jnp.int32, key.shape, 1) & m) == 0
    if K < _C:
        dird = (lax.broadcasted_iota(jnp.int32, key.shape, 2) & K) == 0
    else:
        dird = (lax.broadcasted_iota(jnp.int32, key.shape, 1) & (K // _C)) == 0
    cmp = (key > kp) | ((key == kp) & (idx < xp))
    take_own = cmp == (lower == dird)
    return jnp.where(take_own, key, kp), jnp.where(take_own, idx, xp)


def _sort_body(s_ref, si_ref):
    key = s_ref[...]
    idx = (lax.broadcasted_iota(jnp.int32, key.shape, 1) * _C +
           lax.broadcasted_iota(jnp.int32, key.shape, 2))
    for t in range(1, 13):
        K = 2 ** t
        j = K // 2
        while j >= 1:
            key, idx = _sort_stage(key, idx, K, j)
            j //= 2
    si_ref[...] = idx


def _sort(scores3):
    b = scores3.shape[0]
    return pl.pallas_call(
        _sort_body,
        in_specs=[pl.BlockSpec((b, _R, _C), lambda: (0, 0, 0))],
        out_specs=pl.BlockSpec((b, _R, _C), lambda: (0, 0, 0)),
        out_shape=jax.ShapeDtypeStruct((b, _R, _C), jnp.int32),
    )(scores3)


def _make_sc_invert(b, n):
    mesh = plsc.VectorSubcoreMesh(core_axis_name="c", subcore_axis_name="s")

    @functools.partial(
        pl.kernel,
        mesh=mesh,
        out_type=jax.ShapeDtypeStruct((b, n), jnp.int32),
        scratch_types=[
            pltpu.VMEM((n,), jnp.int32),
            pltpu.VMEM((n,), jnp.int32),
        ],
        compiler_params=pltpu.CompilerParams(needs_layout_passes=False),
    )
    def sc_invert(sidx_hbm, out_hbm, sidx_v, row_v):
        wid = lax.axis_index("s") * 2 + lax.axis_index("c")

        @pl.when(wid < b)
        def _():
            pltpu.sync_copy(sidx_hbm.at[wid], sidx_v)

            def body(v, carry):
                tok = sidx_v[pl.ds(v * 16, 16)]
                pos = lax.iota(jnp.int32, 16) + v * 16
                plsc.store_scatter(row_v, [tok], pos)
                return carry

            lax.fori_loop(0, n // 16, body, 0)
            pltpu.sync_copy(row_v, out_hbm.at[wid])

    return sc_invert


def _add_body(f_ref, r_ref, e_ref, o_ref, *, k):
    x = f_ref[0]
    r = r_ref[0]
    e = e_ref[0]
    o_ref[0] = x + jnp.where(r < k, 1.0, 0.0) * e


def _masked_add(feat, ranks_col, emb, k, tblk):
    b, n, c = feat.shape
    body = functools.partial(_add_body, k=k)
    return pl.pallas_call(
        body,
        grid=(b, n // tblk),
        in_specs=[
            pl.BlockSpec((1, tblk, c), lambda i, t: (i, t, 0)),
            pl.BlockSpec((1, tblk, 1), lambda i, t: (i, t, 0)),
            pl.BlockSpec((1, 1, c), lambda i, t: (0, 0, 0)),
        ],
        out_specs=pl.BlockSpec((1, tblk, c), lambda i, t: (i, t, 0)),
        out_shape=jax.ShapeDtypeStruct((b, n, c), jnp.float32),
    )(feat, ranks_col, emb)


def kernel(feat, emb):
    b, n, c = feat.shape
    k = int(0.4 * n)

    sorted_idx = _scores_sort(feat, tblk=2048)
    sorted_idx = sorted_idx.reshape(b, n)
    indices = sorted_idx[:, :k]
    ranks = _make_sc_invert(b, n)(sorted_idx)
    feat_out = _masked_add(feat, ranks.reshape(b, n, 1), emb, k, tblk=1024)
    return (feat_out, indices)
